# baseline (speedup 1.0000x reference)
import functools

import jax
import jax.numpy as jnp
from jax import lax
from jax.experimental import pallas as pl
from jax.experimental.pallas import tpu as pltpu
from jax.experimental.pallas import tpu_sc as plsc

N = 10000
P = 10240
E = 320000
D = 128
DH = 64
NC, NS = 2, 16
K = 125
NB = E // (NS * K)
NBD = E // (NC * NS * K)
RPT = P // NS

_mesh = plsc.VectorSubcoreMesh(core_axis_name="c", subcore_axis_name="s")

_f32 = jnp.float32


def _zero_fill(buf, n_rows, width):
    def body(i, _):
        for j in range(width // 16):
            buf[i, pl.ds(j * 16, 16)] = jnp.zeros((16,), _f32)
        return 0
    lax.fori_loop(0, n_rows, body, 0, unroll=2)


@functools.partial(
    pl.kernel,
    out_type=[jax.ShapeDtypeStruct((NC, P), _f32),
              jax.ShapeDtypeStruct((NC, P), _f32)],
    mesh=_mesh,
    scratch_types=[
        pltpu.VMEM((NB, K), jnp.int32),
        pltpu.VMEM((NB, K), jnp.int32),
        pltpu.VMEM((1, 128), _f32),
        pltpu.VMEM((5, 128), _f32),
        pltpu.VMEM((5, 128), _f32),
        pltpu.VMEM((NBD, K), _f32),
        pltpu.VMEM_SHARED((P,), _f32),
        pltpu.VMEM_SHARED((P,), _f32),
        pltpu.VMEM_SHARED((P,), _f32),
        pltpu.SemaphoreType.DMA,
        pltpu.SemaphoreType.DMA,
        pltpu.SemaphoreType.DMA,
    ],
)
def _prep_kernel(e_hbm, dinv_out, s_out, src_v, dst_v, ones_v,
                 deg_v, dinv_v, vals_v, acc_sh, s_sh, dinv_sh, dsem, sgsem,
                 sssem):
    c = lax.axis_index("c")
    s = lax.axis_index("s")
    pltpu.sync_copy(e_hbm.at[0, s], src_v)
    pltpu.sync_copy(e_hbm.at[1, s], dst_v)
    for j in range(8):
        ones_v[0, pl.ds(j * 16, 16)] = jnp.full((16,), 1.0, _f32)
        deg_v[0, pl.ds(j * 16, 16)] = jnp.zeros((16,), _f32)
    for k in range(5):
        pltpu.sync_copy(deg_v.at[0], acc_sh.at[pl.ds(s * RPT + k * 128, 128)])
        pltpu.sync_copy(deg_v.at[0], s_sh.at[pl.ds(s * RPT + k * 128, 128)])
    plsc.subcore_barrier()

    def dfire(j, _):
        pltpu.async_copy(ones_v.at[0, pl.ds(0, K)], acc_sh.at[dst_v.at[j]],
                         dsem, add=True)
        return 0
    lax.fori_loop(0, NB, dfire, 0)

    def ddrain(j, _):
        pltpu.make_async_copy(ones_v.at[0, pl.ds(0, K)],
                              acc_sh.at[dst_v.at[0]], dsem).wait()
        return 0
    lax.fori_loop(0, NB, ddrain, 0)
    plsc.subcore_barrier()

    for k in range(5):
        pltpu.sync_copy(acc_sh.at[pl.ds(s * RPT + k * 128, 128)],
                        deg_v.at[k])
    for k in range(5):
        for j in range(8):
            x = deg_v[k, pl.ds(j * 16, 16)] + 1.0
            y = 1.0 / x
            hx = 0.5 * x
            for _ in range(24):
                y = y * (1.5 - hx * y * y)
            dinv_v[k, pl.ds(j * 16, 16)] = y
    for k in range(5):
        pltpu.sync_copy(dinv_v.at[k],
                        dinv_out.at[c, pl.ds(s * RPT + k * 128, 128)])
        pltpu.sync_copy(dinv_v.at[k],
                        dinv_sh.at[pl.ds(s * RPT + k * 128, 128)])
    plsc.subcore_barrier()

    def sgather(j, _):
        pltpu.async_copy(dinv_sh.at[dst_v.at[c * NBD + j]], vals_v.at[j],
                         sgsem)
        return 0
    lax.fori_loop(0, NBD, sgather, 0)

    def sgdrain(j, _):
        pltpu.make_async_copy(dinv_sh.at[dst_v.at[0]], vals_v.at[0],
                              sgsem).wait()
        return 0
    lax.fori_loop(0, NBD, sgdrain, 0)

    def sfire(j, _):
        pltpu.async_copy(vals_v.at[j], s_sh.at[src_v.at[c * NBD + j]],
                         sssem, add=True)
        return 0
    lax.fori_loop(0, NBD, sfire, 0)

    def ssdrain(j, _):
        pltpu.make_async_copy(vals_v.at[0], s_sh.at[src_v.at[0]],
                              sssem).wait()
        return 0
    lax.fori_loop(0, NBD, ssdrain, 0)
    plsc.subcore_barrier()
    pltpu.sync_copy(s_sh.at[pl.ds(s * RPT, RPT)],
                    s_out.at[c, pl.ds(s * RPT, RPT)])


NBS = NBD


def _make_edge_kernel(with_s, nbuf):
    del with_s, nbuf
    HB = NB // 2
    out_type = [jax.ShapeDtypeStruct((NC, P, DH), _f32)]
    scratch = [
        pltpu.VMEM((HB, K), jnp.int32),
        pltpu.VMEM((HB, K), jnp.int32),
        pltpu.VMEM((2, K, DH), _f32),
        pltpu.VMEM((64, 64), _f32),
        pltpu.VMEM_SHARED((P, DH), _f32),
        pltpu.VMEM_SHARED((P, DH), _f32),
        pltpu.SemaphoreType.DMA((2,)),
        pltpu.SemaphoreType.DMA((2,)),
        pltpu.SemaphoreType.DMA,
    ]

    def body(g_hbm, e_hbm, r_out, src_v, dst_v, rows_v, zer_v, acc_sh,
             g_sh, gsem, ssem, stsem):
        c = lax.axis_index("c")
        s = lax.axis_index("s")
        pltpu.async_copy(g_hbm.at[c, pl.ds(s * (N // NS), N // NS)],
                         g_sh.at[pl.ds(s * (N // NS), N // NS)], stsem)
        pltpu.sync_copy(e_hbm.at[0, s, pl.ds(0, HB)], src_v)
        pltpu.sync_copy(e_hbm.at[1, s, pl.ds(0, HB)], dst_v)
        _zero_fill(zer_v, 64, 64)
        for k in range(10):
            pltpu.sync_copy(zer_v, acc_sh.at[pl.ds(s * RPT + k * 64, 64)])
        pltpu.make_async_copy(g_hbm.at[c, pl.ds(0, N // NS)],
                              g_sh.at[pl.ds(0, N // NS)], stsem).wait()
        plsc.subcore_barrier()

        for hh in range(2):
            if hh == 1:
                pltpu.sync_copy(e_hbm.at[0, s, pl.ds(HB, HB)], src_v)
                pltpu.sync_copy(e_hbm.at[1, s, pl.ds(HB, HB)], dst_v)
            for u in range(2):
                pltpu.async_copy(g_sh.at[src_v.at[u]], rows_v.at[u],
                                 gsem.at[u])

            def loop(i, _):
                for u in range(2):
                    j = i * 2 + u
                    pltpu.make_async_copy(g_sh.at[src_v.at[j]],
                                          rows_v.at[u], gsem.at[u]).wait()
                    pltpu.async_copy(rows_v.at[u], acc_sh.at[dst_v.at[j]],
                                     ssem.at[u], add=True)
                for u in range(2):
                    j = i * 2 + u
                    pltpu.make_async_copy(rows_v.at[u],
                                          acc_sh.at[dst_v.at[j]],
                                          ssem.at[u]).wait()

                    @pl.when(i < HB // 2 - 1)
                    def _():
                        jn = (i + 1) * 2 + u
                        pltpu.async_copy(g_sh.at[src_v.at[jn]],
                                         rows_v.at[u], gsem.at[u])
                return 0
            lax.fori_loop(0, HB // 2, loop, 0)
        plsc.subcore_barrier()
        for k in range(5):
            pltpu.sync_copy(acc_sh.at[pl.ds(s * RPT + k * 128, 128)],
                            r_out.at[c, pl.ds(s * RPT + k * 128, 128)])

    return pl.kernel(body, out_type=out_type, mesh=_mesh,
                     scratch_types=scratch,
                     compiler_params=pltpu.CompilerParams(
                         use_tc_tiling_on_sc=False))


_edge_kernel = _make_edge_kernel(False, 5)


BR = 2000
GRID = N // BR


def _split(t, dinv, out_ref):
    out_ref[0] = t[:, :DH] * dinv
    out_ref[1] = t[:, DH:] * dinv


def _tc1_body(x_ref, w1_ref, dv_ref, g1_ref):
    dinv = dv_ref[...][0]
    t = jnp.dot(x_ref[...], w1_ref[...], preferred_element_type=_f32,
                precision=lax.Precision.DEFAULT)
    _split(t, dinv, g1_ref)


_tc1 = pl.pallas_call(
    _tc1_body,
    grid=(GRID,),
    in_specs=[
        pl.BlockSpec((BR, D), lambda i: (i, 0)),
        pl.BlockSpec((D, D), lambda i: (0, 0)),
        pl.BlockSpec((1, BR, 1), lambda i: (0, i, 0)),
    ],
    out_specs=pl.BlockSpec((2, BR, DH), lambda i: (0, i, 0)),
    out_shape=jax.ShapeDtypeStruct((NC, N, DH), _f32),
)


def _tc2_body(rp_ref, g1_ref, dv_ref, b1_ref, w2_ref, g2_ref):
    rp = rp_ref[...]
    g1 = g1_ref[...]
    dinv = dv_ref[...][0]
    r = jnp.concatenate([rp[0] + g1[0], rp[1] + g1[1]], axis=1)
    h1 = jax.nn.relu(dinv * r + b1_ref[...])
    t = jnp.dot(h1, w2_ref[...], preferred_element_type=_f32,
                precision=lax.Precision.DEFAULT)
    _split(t, dinv, g2_ref)


_tc2 = pl.pallas_call(
    _tc2_body,
    grid=(GRID,),
    in_specs=[
        pl.BlockSpec((2, BR, DH), lambda i: (0, i, 0)),
        pl.BlockSpec((2, BR, DH), lambda i: (0, i, 0)),
        pl.BlockSpec((1, BR, 1), lambda i: (0, i, 0)),
        pl.BlockSpec((1, D), lambda i: (0, 0)),
        pl.BlockSpec((D, D), lambda i: (0, 0)),
    ],
    out_specs=pl.BlockSpec((2, BR, DH), lambda i: (0, i, 0)),
    out_shape=jax.ShapeDtypeStruct((NC, N, DH), _f32),
)


def _tc3_body(rp_ref, g2_ref, dv_ref, b2_ref, sp_ref, w3_ref, b3_ref,
              out_ref, acc_ref):
    i = pl.program_id(0)
    rp = rp_ref[...]
    g2 = g2_ref[...]
    dinv = dv_ref[...][0]
    r = jnp.concatenate([rp[0] + g2[0], rp[1] + g2[1]], axis=1)
    h2 = jax.nn.relu(dinv * r + b2_ref[...])
    sp = sp_ref[...]
    w = dinv * (sp[0] + sp[1] + dinv)
    contrib = jnp.sum(w * h2, axis=0, keepdims=True)

    @pl.when(i == 0)
    def _():
        acc_ref[...] = contrib

    @pl.when(i > 0)
    def _():
        acc_ref[...] = acc_ref[...] + contrib

    @pl.when(i == GRID - 1)
    def _():
        u = acc_ref[...] * (1.0 / N)
        out_ref[...] = jnp.dot(u, w3_ref[...], preferred_element_type=_f32,
                               precision=lax.Precision.DEFAULT) + b3_ref[...]


_tc3 = pl.pallas_call(
    _tc3_body,
    grid=(GRID,),
    in_specs=[
        pl.BlockSpec((2, BR, DH), lambda i: (0, i, 0)),
        pl.BlockSpec((2, BR, DH), lambda i: (0, i, 0)),
        pl.BlockSpec((1, BR, 1), lambda i: (0, i, 0)),
        pl.BlockSpec((1, D), lambda i: (0, 0)),
        pl.BlockSpec((2, BR, 1), lambda i: (0, i, 0)),
        pl.BlockSpec((D, 64), lambda i: (0, 0)),
        pl.BlockSpec((1, 64), lambda i: (0, 0)),
    ],
    out_specs=pl.BlockSpec((1, 64), lambda i: (0, 0)),
    out_shape=jax.ShapeDtypeStruct((1, 64), _f32),
    scratch_shapes=[pltpu.VMEM((1, D), _f32)],
)


def kernel(x, edge_index, W1, b1, W2, b2, W3, b3):
    ei = edge_index.astype(jnp.int32).reshape(2, NS, NB, K)
    dinv_p, s_raw = _prep_kernel(ei)
    dv = dinv_p.reshape(NC, P, 1)
    g1 = _tc1(x, W1, dv)
    (r1,) = _edge_kernel(g1, ei)
    g2 = _tc2(r1, g1, dv, b1.reshape(1, D), W2)
    (r2,) = _edge_kernel(g2, ei)
    out = _tc3(r2, g2, dv, b2.reshape(1, D), s_raw.reshape(NC, P, 1),
               W3, b3.reshape(1, 64))
    return out


# speedup vs baseline: 1.2550x; 1.2550x over previous
import functools

import jax
import jax.numpy as jnp
from jax import lax
from jax.experimental import pallas as pl
from jax.experimental.pallas import tpu as pltpu
from jax.experimental.pallas import tpu_sc as plsc

N = 10000
P = 10240
E = 320000
D = 128
DH = 64
NC, NS = 2, 16
K = 125
NB = E // (NS * K)
NBD = E // (NC * NS * K)
RPT = P // NS

_mesh = plsc.VectorSubcoreMesh(core_axis_name="c", subcore_axis_name="s")

_f32 = jnp.float32


def _zero_fill(buf, n_rows, width):
    def body(i, _):
        for j in range(width // 16):
            buf[i, pl.ds(j * 16, 16)] = jnp.zeros((16,), _f32)
        return 0
    lax.fori_loop(0, n_rows, body, 0, unroll=2)


@functools.partial(
    pl.kernel,
    out_type=[jax.ShapeDtypeStruct((NC, P), _f32),
              jax.ShapeDtypeStruct((NC, P), _f32)],
    mesh=_mesh,
    scratch_types=[
        pltpu.VMEM((NB, K), jnp.int32),
        pltpu.VMEM((NB, K), jnp.int32),
        pltpu.VMEM((1, 128), _f32),
        pltpu.VMEM((5, 128), _f32),
        pltpu.VMEM((5, 128), _f32),
        pltpu.VMEM((NBD, K), _f32),
        pltpu.VMEM_SHARED((P,), _f32),
        pltpu.VMEM_SHARED((P,), _f32),
        pltpu.VMEM_SHARED((P,), _f32),
        pltpu.SemaphoreType.DMA,
        pltpu.SemaphoreType.DMA,
        pltpu.SemaphoreType.DMA,
    ],
)
def _prep_kernel(e_hbm, dinv_out, s_out, src_v, dst_v, ones_v,
                 deg_v, dinv_v, vals_v, acc_sh, s_sh, dinv_sh, dsem, sgsem,
                 sssem):
    c = lax.axis_index("c")
    s = lax.axis_index("s")
    pltpu.sync_copy(e_hbm.at[0, s], src_v)
    pltpu.sync_copy(e_hbm.at[1, s], dst_v)
    for j in range(8):
        ones_v[0, pl.ds(j * 16, 16)] = jnp.full((16,), 1.0, _f32)
        deg_v[0, pl.ds(j * 16, 16)] = jnp.zeros((16,), _f32)
    for k in range(5):
        pltpu.sync_copy(deg_v.at[0], acc_sh.at[pl.ds(s * RPT + k * 128, 128)])
        pltpu.sync_copy(deg_v.at[0], s_sh.at[pl.ds(s * RPT + k * 128, 128)])
    plsc.subcore_barrier()

    def dfire(j, _):
        pltpu.async_copy(ones_v.at[0, pl.ds(0, K)], acc_sh.at[dst_v.at[j]],
                         dsem, add=True)
        return 0
    lax.fori_loop(0, NB, dfire, 0)

    def ddrain(j, _):
        pltpu.make_async_copy(ones_v.at[0, pl.ds(0, K)],
                              acc_sh.at[dst_v.at[0]], dsem).wait()
        return 0
    lax.fori_loop(0, NB, ddrain, 0)
    plsc.subcore_barrier()

    for k in range(5):
        pltpu.sync_copy(acc_sh.at[pl.ds(s * RPT + k * 128, 128)],
                        deg_v.at[k])
    for k in range(5):
        for j in range(8):
            x = deg_v[k, pl.ds(j * 16, 16)] + 1.0
            y = 1.0 / x
            hx = 0.5 * x
            for _ in range(24):
                y = y * (1.5 - hx * y * y)
            dinv_v[k, pl.ds(j * 16, 16)] = y
    for k in range(5):
        pltpu.sync_copy(dinv_v.at[k],
                        dinv_out.at[c, pl.ds(s * RPT + k * 128, 128)])
        pltpu.sync_copy(dinv_v.at[k],
                        dinv_sh.at[pl.ds(s * RPT + k * 128, 128)])
    plsc.subcore_barrier()

    def sgather(j, _):
        pltpu.async_copy(dinv_sh.at[dst_v.at[c * NBD + j]], vals_v.at[j],
                         sgsem)
        return 0
    lax.fori_loop(0, NBD, sgather, 0)

    def sgdrain(j, _):
        pltpu.make_async_copy(dinv_sh.at[dst_v.at[0]], vals_v.at[0],
                              sgsem).wait()
        return 0
    lax.fori_loop(0, NBD, sgdrain, 0)

    def sfire(j, _):
        pltpu.async_copy(vals_v.at[j], s_sh.at[src_v.at[c * NBD + j]],
                         sssem, add=True)
        return 0
    lax.fori_loop(0, NBD, sfire, 0)

    def ssdrain(j, _):
        pltpu.make_async_copy(vals_v.at[0], s_sh.at[src_v.at[0]],
                              sssem).wait()
        return 0
    lax.fori_loop(0, NBD, ssdrain, 0)
    plsc.subcore_barrier()
    pltpu.sync_copy(s_sh.at[pl.ds(s * RPT, RPT)],
                    s_out.at[c, pl.ds(s * RPT, RPT)])


NBS = NBD


def _make_edge_kernel(with_s, nbuf):
    nrounds = NB // nbuf
    out_type = [jax.ShapeDtypeStruct((NC, P, DH), _f32)]
    scratch = [
        pltpu.VMEM((NB, K), jnp.int32),
        pltpu.VMEM((NB, K), jnp.int32),
        pltpu.VMEM((nbuf, K, DH), _f32),
        pltpu.VMEM((64, 64), _f32),
        pltpu.VMEM_SHARED((P, DH), _f32),
        pltpu.SemaphoreType.DMA((nbuf,)),
        pltpu.SemaphoreType.DMA((nbuf,)),
    ]
    if with_s:
        out_type.append(jax.ShapeDtypeStruct((NC, P), _f32))
        scratch += [
            pltpu.VMEM((NBS, K), _f32),
            pltpu.VMEM_SHARED((P,), _f32),
            pltpu.SemaphoreType.DMA,
            pltpu.SemaphoreType.DMA,
        ]

    def body(g_hbm, e_hbm, *refs):
        if with_s:
            (r_out, s_out, src_v, dst_v, rows_v, zer_v, acc_sh, gsem, ssem,
             vals_v, s_sh, sgsem, sssem) = refs
        else:
            (r_out, src_v, dst_v, rows_v, zer_v, acc_sh, gsem,
             ssem) = refs
        c = lax.axis_index("c")
        s = lax.axis_index("s")
        pltpu.sync_copy(e_hbm.at[0, s], src_v)
        pltpu.sync_copy(e_hbm.at[1, s], dst_v)
        _zero_fill(zer_v, 64, 64)
        for k in range(10):
            pltpu.sync_copy(zer_v, acc_sh.at[pl.ds(s * RPT + k * 64, 64)])
        if with_s:
            for k in range(10):
                pltpu.sync_copy(zer_v.at[0],
                                s_sh.at[pl.ds(s * RPT + k * 64, 64)])
        plsc.subcore_barrier()

        for u in range(nbuf):
            pltpu.async_copy(g_hbm.at[c].at[src_v.at[u]], rows_v.at[u],
                             gsem.at[u])

        if with_s:
            def sgather(j, _):
                pltpu.async_copy(dinv_hbm.at[dst_v.at[c * NBS + j]],
                                 vals_v.at[j], sgsem)
                return 0
            lax.fori_loop(0, NBS, sgather, 0)

        def loop(i, _):
            for u in range(nbuf):
                j = i * nbuf + u
                pltpu.make_async_copy(g_hbm.at[c].at[src_v.at[j]],
                                      rows_v.at[u], gsem.at[u]).wait()
                pltpu.async_copy(rows_v.at[u], acc_sh.at[dst_v.at[j]],
                                 ssem.at[u], add=True)
            for u in range(nbuf):
                j = i * nbuf + u
                pltpu.make_async_copy(rows_v.at[u], acc_sh.at[dst_v.at[j]],
                                      ssem.at[u]).wait()

                @pl.when(i < nrounds - 1)
                def _():
                    jn = (i + 1) * nbuf + u
                    pltpu.async_copy(g_hbm.at[c].at[src_v.at[jn]],
                                     rows_v.at[u], gsem.at[u])
            if with_s:
                @pl.when(i == nrounds - 5)
                def _():
                    def sdrain_g(j, _):
                        pltpu.make_async_copy(dinv_hbm.at[dst_v.at[c * NBS]],
                                              vals_v.at[0], sgsem).wait()
                        return 0
                    lax.fori_loop(0, NBS, sdrain_g, 0)

                    def sfire(j, _):
                        pltpu.async_copy(vals_v.at[j],
                                         s_sh.at[src_v.at[c * NBS + j]],
                                         sssem, add=True)
                        return 0
                    lax.fori_loop(0, NBS, sfire, 0)
            return 0
        lax.fori_loop(0, nrounds, loop, 0)

        if with_s:
            def sdrain_s(j, _):
                pltpu.make_async_copy(vals_v.at[0], s_sh.at[src_v.at[c * NBS]],
                                      sssem).wait()
                return 0
            lax.fori_loop(0, NBS, sdrain_s, 0)
        plsc.subcore_barrier()
        for k in range(5):
            pltpu.sync_copy(acc_sh.at[pl.ds(s * RPT + k * 128, 128)],
                            r_out.at[c, pl.ds(s * RPT + k * 128, 128)])
        if with_s:
            pltpu.sync_copy(s_sh.at[pl.ds(s * RPT, RPT)],
                            s_out.at[c, pl.ds(s * RPT, RPT)])

    return pl.kernel(body, out_type=out_type, mesh=_mesh,
                     scratch_types=scratch,
                     compiler_params=pltpu.CompilerParams(
                         use_tc_tiling_on_sc=False))


_edge_kernel = _make_edge_kernel(False, 5)


BR = 2000
GRID = N // BR


def _split(t, dinv, out_ref):
    out_ref[0] = t[:, :DH] * dinv
    out_ref[1] = t[:, DH:] * dinv


def _tc1_body(x_ref, w1_ref, dv_ref, g1_ref):
    dinv = dv_ref[...][0]
    t = jnp.dot(x_ref[...], w1_ref[...], preferred_element_type=_f32,
                precision=lax.Precision.DEFAULT)
    _split(t, dinv, g1_ref)


_tc1 = pl.pallas_call(
    _tc1_body,
    grid=(GRID,),
    in_specs=[
        pl.BlockSpec((BR, D), lambda i: (i, 0)),
        pl.BlockSpec((D, D), lambda i: (0, 0)),
        pl.BlockSpec((1, BR, 1), lambda i: (0, i, 0)),
    ],
    out_specs=pl.BlockSpec((2, BR, DH), lambda i: (0, i, 0)),
    out_shape=jax.ShapeDtypeStruct((NC, N, DH), _f32),
)


def _tc2_body(rp_ref, g1_ref, dv_ref, b1_ref, w2_ref, g2_ref):
    rp = rp_ref[...]
    g1 = g1_ref[...]
    dinv = dv_ref[...][0]
    r = jnp.concatenate([rp[0] + g1[0], rp[1] + g1[1]], axis=1)
    h1 = jax.nn.relu(dinv * r + b1_ref[...])
    t = jnp.dot(h1, w2_ref[...], preferred_element_type=_f32,
                precision=lax.Precision.DEFAULT)
    _split(t, dinv, g2_ref)


_tc2 = pl.pallas_call(
    _tc2_body,
    grid=(GRID,),
    in_specs=[
        pl.BlockSpec((2, BR, DH), lambda i: (0, i, 0)),
        pl.BlockSpec((2, BR, DH), lambda i: (0, i, 0)),
        pl.BlockSpec((1, BR, 1), lambda i: (0, i, 0)),
        pl.BlockSpec((1, D), lambda i: (0, 0)),
        pl.BlockSpec((D, D), lambda i: (0, 0)),
    ],
    out_specs=pl.BlockSpec((2, BR, DH), lambda i: (0, i, 0)),
    out_shape=jax.ShapeDtypeStruct((NC, N, DH), _f32),
)


def _tc3_body(rp_ref, g2_ref, dv_ref, b2_ref, sp_ref, w3_ref, b3_ref,
              out_ref, acc_ref):
    i = pl.program_id(0)
    rp = rp_ref[...]
    g2 = g2_ref[...]
    dinv = dv_ref[...][0]
    r = jnp.concatenate([rp[0] + g2[0], rp[1] + g2[1]], axis=1)
    h2 = jax.nn.relu(dinv * r + b2_ref[...])
    sp = sp_ref[...]
    w = dinv * (sp[0] + sp[1] + dinv)
    contrib = jnp.sum(w * h2, axis=0, keepdims=True)

    @pl.when(i == 0)
    def _():
        acc_ref[...] = contrib

    @pl.when(i > 0)
    def _():
        acc_ref[...] = acc_ref[...] + contrib

    @pl.when(i == GRID - 1)
    def _():
        u = acc_ref[...] * (1.0 / N)
        out_ref[...] = jnp.dot(u, w3_ref[...], preferred_element_type=_f32,
                               precision=lax.Precision.DEFAULT) + b3_ref[...]


_tc3 = pl.pallas_call(
    _tc3_body,
    grid=(GRID,),
    in_specs=[
        pl.BlockSpec((2, BR, DH), lambda i: (0, i, 0)),
        pl.BlockSpec((2, BR, DH), lambda i: (0, i, 0)),
        pl.BlockSpec((1, BR, 1), lambda i: (0, i, 0)),
        pl.BlockSpec((1, D), lambda i: (0, 0)),
        pl.BlockSpec((2, BR, 1), lambda i: (0, i, 0)),
        pl.BlockSpec((D, 64), lambda i: (0, 0)),
        pl.BlockSpec((1, 64), lambda i: (0, 0)),
    ],
    out_specs=pl.BlockSpec((1, 64), lambda i: (0, 0)),
    out_shape=jax.ShapeDtypeStruct((1, 64), _f32),
    scratch_shapes=[pltpu.VMEM((1, D), _f32)],
)


def kernel(x, edge_index, W1, b1, W2, b2, W3, b3):
    ei = edge_index.astype(jnp.int32).reshape(2, NS, NB, K)
    dinv_p, s_raw = _prep_kernel(ei)
    dv = dinv_p.reshape(NC, P, 1)
    g1 = _tc1(x, W1, dv)
    (r1,) = _edge_kernel(g1, ei)
    g2 = _tc2(r1, g1, dv, b1.reshape(1, D), W2)
    (r2,) = _edge_kernel(g2, ei)
    out = _tc3(r2, g2, dv, b2.reshape(1, D), s_raw.reshape(NC, P, 1),
               W3, b3.reshape(1, 64))
    return out
